# baseline (speedup 1.0000x reference)
import functools

import jax
import jax.numpy as jnp
from jax import lax
from jax.experimental import pallas as pl
from jax.experimental.pallas import tpu as pltpu
from jax.experimental.pallas import tpu_sc as plsc

N = 50000
E = 800000
G = 2048
F = 128
NB = 16
CB = 8
EPS = 1e-5

E_PAD = 802816
PAD_E = E_PAD - E
CHUNKS = 392
NHALF = 25000
N_ACC = 25008
N_POOL = 53248
G_ACC = 2176
BN = 400

NCORES = 2
_mesh = plsc.VectorSubcoreMesh(core_axis_name="c", subcore_axis_name="s",
                               num_cores=NCORES)

NBUF = 8
GRP = CHUNKS // NBUF


@functools.partial(
    pl.kernel,
    out_type=jax.ShapeDtypeStruct((NB, N, CB), jnp.float32),
    mesh=_mesh,
    compiler_params=pltpu.CompilerParams(use_tc_tiling_on_sc=False),
    scratch_types=[
        pltpu.VMEM((CHUNKS, 128), jnp.int32),
        pltpu.VMEM((CHUNKS, 128), jnp.int32),
        pltpu.VMEM((NBUF, 128, CB), jnp.float32),
        pltpu.VMEM_SHARED((N_ACC, CB), jnp.float32),
        pltpu.SemaphoreType.DMA((NBUF,)),
        pltpu.SemaphoreType.DMA,
    ],
)
def _segsum_sc(y_hbm, src_hbm, dst_hbm, zero_hbm, out_hbm, sidx, didx, bufs,
               acc, sem_g, sem_s):
    c = lax.axis_index("c")
    s = lax.axis_index("s")

    pltpu.sync_copy(src_hbm.at[pl.ds(s * CHUNKS, CHUNKS)], sidx)
    pltpu.sync_copy(dst_hbm.at[pl.ds(s * CHUNKS, CHUNKS)], didx)

    off = c * NHALF

    @pl.loop(0, CHUNKS)
    def _(k):
        for w in range(8):
            d = didx[k, pl.ds(16 * w, 16)] - off
            ok = (d >= 0) & (d < NHALF)
            didx[k, pl.ds(16 * w, 16)] = jnp.where(ok, d, NHALF)

    @pl.loop(0, NB)
    def _(j):
        pltpu.sync_copy(zero_hbm.at[pl.ds(0, 1563)],
                        acc.at[pl.ds(s * 1563, 1563)])
        plsc.subcore_barrier()

        tbl = y_hbm.at[j]

        @pl.loop(0, GRP)
        def _(g):
            k0 = g * NBUF
            gd = [pltpu.async_copy(tbl.at[sidx.at[k0 + b]], bufs.at[b],
                                   sem_g.at[b]) for b in range(NBUF)]
            sd = []
            for b in range(NBUF):
                gd[b].wait()
                sd.append(pltpu.async_copy(bufs.at[b], acc.at[didx.at[k0 + b]],
                                           sem_s, add=True))
            for d in sd:
                d.wait()

        plsc.subcore_barrier()

        @pl.when(s < 8)
        def _():
            pltpu.sync_copy(
                acc.at[pl.ds(s * 3125, 3125)],
                out_hbm.at[j, pl.ds(c * NHALF + s * 3125, 3125)])

        plsc.subcore_barrier()


@functools.partial(
    pl.kernel,
    out_type=jax.ShapeDtypeStruct((NB, G, CB), jnp.float32),
    mesh=_mesh,
    compiler_params=pltpu.CompilerParams(use_tc_tiling_on_sc=False),
    scratch_types=[
        pltpu.VMEM((N_POOL // 128, 128), jnp.int32),
        pltpu.VMEM((NBUF, 128, CB), jnp.float32),
        pltpu.VMEM_SHARED((G_ACC, CB), jnp.float32),
        pltpu.SemaphoreType.DMA((NBUF,)),
        pltpu.SemaphoreType.DMA,
    ],
)
def _pool_sc(h_hbm, bidx_hbm, zero_hbm, out_hbm, pidx, bufs, accp, sem_g,
             sem_s):
    c = lax.axis_index("c")
    s = lax.axis_index("s")

    pltpu.sync_copy(bidx_hbm, pidx)

    @pl.loop(0, NB // NCORES)
    def _(jj):
        j = c * (NB // NCORES) + jj
        pltpu.sync_copy(zero_hbm.at[pl.ds(0, 136)],
                        accp.at[pl.ds(s * 136, 136)])
        plsc.subcore_barrier()

        base = s * 3328

        for g in range(4):
            k0 = g * NBUF
            nb = NBUF if g < 3 else 2
            gd = [pltpu.async_copy(
                      h_hbm.at[j, pl.ds(base + (k0 + b) * 128, 128)],
                      bufs.at[b], sem_g.at[b]) for b in range(nb)]
            sd = []
            for b in range(nb):
                gd[b].wait()
                sd.append(pltpu.async_copy(bufs.at[b],
                                           accp.at[pidx.at[s * 26 + k0 + b]],
                                           sem_s, add=True))
            for d in sd:
                d.wait()

        plsc.subcore_barrier()

        pltpu.sync_copy(accp.at[pl.ds(s * 128, 128)],
                        out_hbm.at[j, pl.ds(s * 128, 128)])
        plsc.subcore_barrier()


def _cat(ref):
    return jnp.concatenate([ref[q] for q in range(NB)], axis=1)


def _split(out_ref, v):
    for q in range(NB):
        out_ref[q] = v[:, q * CB:(q + 1) * CB]


def _p0_body(x_ref, w_ref, out_ref):
    _split(out_ref, jnp.dot(x_ref[...], w_ref[...],
                            preferred_element_type=jnp.float32))


def _p0(x, w1a):
    return pl.pallas_call(
        _p0_body,
        grid=(N // BN,),
        in_specs=[
            pl.BlockSpec((BN, 78), lambda i: (i, 0)),
            pl.BlockSpec((78, F), lambda i: (0, 0)),
        ],
        out_specs=pl.BlockSpec((NB, BN, CB), lambda i: (0, i, 0)),
        out_shape=jax.ShapeDtypeStruct((NB, N_POOL, CB), jnp.float32),
    )(x, w1a)


def _passA_body(y_ref, a_ref, ba_ref, wa_ref, wb_ref, bb_ref, u_ref, sm_ref,
                sq_ref):
    i = pl.program_id(0)
    y = _cat(y_ref)
    a = _cat(a_ref)
    row0 = (lax.broadcasted_iota(jnp.int32, (BN, 1), 0) == 0) & (i == 0)
    a = a - jnp.where(row0, jnp.float32(PAD_E), 0.0) * y
    z = jnp.maximum(
        jnp.dot(y + a, wa_ref[...], preferred_element_type=jnp.float32)
        + ba_ref[...], 0.0)
    t = jnp.dot(z, wb_ref[...], preferred_element_type=jnp.float32) + bb_ref[...]
    u = jnp.maximum(t, 0.0)
    u_ref[...] = u

    @pl.when(i == 0)
    def _():
        sm_ref[...] = jnp.zeros_like(sm_ref)
        sq_ref[...] = jnp.zeros_like(sq_ref)

    sm_ref[...] += jnp.sum(u, axis=0, keepdims=True)
    sq_ref[...] += jnp.sum(u * u, axis=0, keepdims=True)


def _passA(ycb, acb, ba, wa, wb, bb):
    return pl.pallas_call(
        _passA_body,
        grid=(N // BN,),
        in_specs=[
            pl.BlockSpec((NB, BN, CB), lambda i: (0, i, 0)),
            pl.BlockSpec((NB, BN, CB), lambda i: (0, i, 0)),
            pl.BlockSpec((1, F), lambda i: (0, 0)),
            pl.BlockSpec((F, F), lambda i: (0, 0)),
            pl.BlockSpec((F, F), lambda i: (0, 0)),
            pl.BlockSpec((1, F), lambda i: (0, 0)),
        ],
        out_specs=[
            pl.BlockSpec((BN, F), lambda i: (i, 0)),
            pl.BlockSpec((1, F), lambda i: (0, 0)),
            pl.BlockSpec((1, F), lambda i: (0, 0)),
        ],
        out_shape=[
            jax.ShapeDtypeStruct((N, F), jnp.float32),
            jax.ShapeDtypeStruct((1, F), jnp.float32),
            jax.ShapeDtypeStruct((1, F), jnp.float32),
        ],
    )(ycb, acb, ba, wa, wb, bb)


def _bn_affine(sm, sq, gam, bet):
    m = sm * (1.0 / N)
    var = sq * (1.0 / N) - m * m
    sc = gam / jnp.sqrt(var + EPS)
    off = bet - m * sc
    return sc, off


def _passBaff_body(u_ref, sm_ref, sq_ref, g_ref, be_ref, out_ref):
    sc, off = _bn_affine(sm_ref[...], sq_ref[...], g_ref[...], be_ref[...])
    _split(out_ref, u_ref[...] * sc + off)


def _passB_aff(u, sm, sq, gam, bet):
    return pl.pallas_call(
        _passBaff_body,
        grid=(N // BN,),
        in_specs=[
            pl.BlockSpec((BN, F), lambda i: (i, 0)),
            pl.BlockSpec((1, F), lambda i: (0, 0)),
            pl.BlockSpec((1, F), lambda i: (0, 0)),
            pl.BlockSpec((1, F), lambda i: (0, 0)),
            pl.BlockSpec((1, F), lambda i: (0, 0)),
        ],
        out_specs=pl.BlockSpec((NB, BN, CB), lambda i: (0, i, 0)),
        out_shape=jax.ShapeDtypeStruct((NB, N_POOL, CB), jnp.float32),
    )(u, sm, sq, gam, bet)


def _final_body(p_ref, w_ref, b_ref, out_ref):
    hg = _cat(p_ref)
    out_ref[...] = jnp.maximum(
        jnp.dot(hg, w_ref[...], preferred_element_type=jnp.float32) + b_ref[...],
        0.0)


def _final(pool, wfc, bfc):
    return pl.pallas_call(
        _final_body,
        grid=(1,),
        in_specs=[
            pl.BlockSpec((NB, G, CB), lambda i: (0, 0, 0)),
            pl.BlockSpec((F, F), lambda i: (0, 0)),
            pl.BlockSpec((1, F), lambda i: (0, 0)),
        ],
        out_specs=pl.BlockSpec((G, F), lambda i: (0, 0)),
        out_shape=jax.ShapeDtypeStruct((G, F), jnp.float32),
    )(pool, wfc, bfc)


def _branch_body(f_ref, w_ref, b_ref, g_ref, be_ref, out_ref):
    v = jnp.maximum(
        jnp.dot(f_ref[...], w_ref[...], preferred_element_type=jnp.float32)
        + b_ref[...], 0.0)
    m = jnp.mean(v, axis=0, keepdims=True)
    var = jnp.mean(v * v, axis=0, keepdims=True) - m * m
    out_ref[...] = (v - m) / jnp.sqrt(var + EPS) * g_ref[...] + be_ref[...]


def _branch(feat, w, b, gam, bet):
    nr, dk = feat.shape
    return pl.pallas_call(
        _branch_body,
        grid=(1,),
        in_specs=[
            pl.BlockSpec((nr, dk), lambda i: (0, 0)),
            pl.BlockSpec((dk, F), lambda i: (0, 0)),
            pl.BlockSpec((1, F), lambda i: (0, 0)),
            pl.BlockSpec((1, F), lambda i: (0, 0)),
            pl.BlockSpec((1, F), lambda i: (0, 0)),
        ],
        out_specs=pl.BlockSpec((nr, F), lambda i: (0, 0)),
        out_shape=jax.ShapeDtypeStruct((nr, F), jnp.float32),
    )(feat, w, b, gam, bet)


def kernel(x, edge_index, batch, mic_feature, dis_feature, params):
    p = params
    r2 = lambda v: v.reshape(1, F)

    src = edge_index[0]
    dst = edge_index[1]
    srcp = jnp.concatenate(
        [src, jnp.zeros((E_PAD - E,), jnp.int32)]).reshape(E_PAD // 128, 128)
    dstp = jnp.concatenate(
        [dst, jnp.zeros((E_PAD - E,), jnp.int32)]).reshape(E_PAD // 128, 128)
    batchp = jnp.concatenate(
        [batch, jnp.full((N_POOL - N,), G, jnp.int32)]).reshape(N_POOL // 128, 128)
    del src, dst
    zeros = jnp.zeros((3200, CB), jnp.float32)

    w1a_p = jnp.pad(p['w1a'], ((0, F - 78), (0, 0)))
    ws = (
        jnp.stack([r2(p['b1a']), r2(p['b2a']), r2(p['b3a'])]),
        jnp.stack([w1a_p, p['w2a'], p['w3a']]),
        jnp.stack([p['w1b'], p['w2b'], p['w3b']]),
        jnp.stack([r2(p['b1b']), r2(p['b2b']), r2(p['b3b'])]),
        jnp.stack([r2(p['g1']), r2(p['g2']), r2(p['g3'])]),
        jnp.stack([r2(p['be1']), r2(p['be2']), r2(p['be3'])]),
    )

    def _layer(y, w):
        ba, wa, wb, bb, gam, bet = w
        a = _segsum_sc(y, srcp, dstp, zeros)
        u, sm, sq = _passA(y, a, ba, wa, wb, bb)
        return _passB_aff(u, sm, sq, gam, bet), None

    xcb = jnp.pad(x, ((0, N_POOL - N), (0, F - 78))).reshape(
        N_POOL, NB, CB).transpose(1, 0, 2)
    h3, _ = lax.scan(_layer, xcb, ws)
    pool = _pool_sc(h3, batchp, zeros)
    hg = _final(pool, p['wfc'], r2(p['bfc']))
    xm = _branch(mic_feature, p['wmic'], r2(p['bmic']), r2(p['gmic']),
                 r2(p['bemic']))
    xd = _branch(dis_feature, p['wdis'], r2(p['bdis']), r2(p['gdis']),
                 r2(p['bedis']))
    return hg, xm, xd
